# baseline (speedup 1.0000x reference)
import jax
import jax.numpy as jnp
from jax import lax
from jax.experimental import pallas as pl
from jax.experimental.pallas import tpu as pltpu
from jax.experimental.pallas import tpu_sc as plsc

_B, _N, _M, _C = 16, 16384, 128, 80
_NC, _NS = 2, 16
_HALF = _N // 2
_K = 16
_S = _HALF // _K
_LW = 160
_LN2 = 0.6931471805599453


def _log_vec(x):
    bits = lax.bitcast_convert_type(x, jnp.int32)
    e = (bits >> 23) - 127
    mant = lax.bitcast_convert_type((bits & 0x7FFFFF) | 0x3F800000,
                                    jnp.float32)
    t = (mant - 1.0) / (mant + 1.0)
    t2 = t * t
    p = 2.0 / 9.0
    p = p * t2 + 2.0 / 7.0
    p = p * t2 + 2.0 / 5.0
    p = p * t2 + 2.0 / 3.0
    p = p * t2 + 2.0
    return e.astype(jnp.float32) * _LN2 + t * p


def _body(boxes_hbm, x1y1_hbm, x2y2_hbm, cls_hbm, flag_hbm, ids_hbm,
          mask_hbm, tconf_hbm, tcls_hbm, tbox_hbm,
          ids_v, clsi_v, lwhi_v, gtp_v, gtq_v, lists_v, cnts_v, wtab_v,
          flag_v, boxes_v, mask_v, tconf_v, tcls_v, tbox_v,
          sem_in, sem_out):
    w = lax.axis_index("s") * _NC + lax.axis_index("c")
    b = w // 2
    h = w % 2
    half0 = h * _HALF

    pltpu.sync_copy(ids_hbm.at[b], ids_v.at[pl.ds(0, _M)])
    pltpu.sync_copy(cls_hbm.at[b], clsi_v.at[pl.ds(0, _M)])
    pltpu.sync_copy(x1y1_hbm.at[b], gtp_v)
    pltpu.sync_copy(x2y2_hbm.at[b], gtq_v)

    lane = lax.iota(jnp.int32, 16)
    zvec = jnp.zeros((16,), jnp.float32)
    zivec = jnp.zeros((16,), jnp.int32)
    e0vec = jnp.where(lane == 0, 1.0, 0.0)
    lane0 = lane == 0
    lane01 = lane < 2

    def prec(g, _):
        sl = pl.ds(g * 16, 16)
        lwhi_v[sl] = _log_vec(gtq_v[sl] - gtp_v[sl])
        return 0
    lax.fori_loop(0, 2 * _M // 16, prec, 0)

    cnts_v[pl.ds(0, 16)] = zivec
    cnts_v[pl.ds(16, 16)] = zivec

    def bldall(m, _):
        idm = ids_v[pl.ds(m, 16)][0]
        rr = idm - half0
        inw = (rr >= 0) & (rr < _HALF)

        @pl.when(inw)
        def _():
            kk = rr >> 9
            cv = cnts_v[pl.ds(kk, 16)][0]
            lv = lists_v[pl.ds(kk * _LW + cv, 16)]
            lists_v[pl.ds(kk * _LW + cv, 16)] = jnp.where(lane0, m, lv)
            cw = cnts_v[pl.ds(kk, 16)]
            cnts_v[pl.ds(kk, 16)] = jnp.where(lane0, cv + 1, cw)
            wv = wtab_v[pl.ds(rr, 16)]
            wtab_v[pl.ds(rr, 16)] = jnp.where(lane0, m, wv)
        return 0
    lax.fori_loop(0, _M, bldall, 0)

    dz = cnts_v[pl.ds(_K, 16)][0] * 0 + cnts_v[pl.ds(_K + 1, 16)][0] * 0

    def initr(r, _):
        o = _C * r
        tcls_v[pl.ds(o, 16)] = e0vec
        tcls_v[pl.ds(o + 16, 16)] = zvec
        tcls_v[pl.ds(o + 32, 16)] = zvec
        tcls_v[pl.ds(o + 48, 16)] = zvec
        tcls_v[pl.ds(o + 64, 16)] = zvec
        return 0
    lax.fori_loop(0, _S, initr, 0)

    def initc(g, _):
        tconf_v[pl.ds(g * 16, 16)] = zvec
        return 0
    lax.fori_loop(0, _S // 16, initc, 0)

    def initz(g, _):
        tbox_v[pl.ds(g * 16, 16)] = zvec
        return 0
    lax.fori_loop(0, 4 * _S // 16, initz, 0)

    def chunk(k, _):
        nlo = half0 + k * _S

        din1 = pltpu.async_copy(flag_hbm.at[b, pl.ds(nlo, _S)],
                                flag_v, sem_in)
        din2 = pltpu.async_copy(boxes_hbm.at[b, pl.ds(4 * nlo, 4 * _S)],
                                boxes_v.at[pl.ds(0, 4 * _S)], sem_in)
        cnt = cnts_v[pl.ds(k, 16)][0]

        din1.wait()
        din2.wait()

        def mk_(g, _):
            sl = pl.ds(g * 16, 16)
            mask_v[sl] = jnp.where(flag_v[sl] > 0, -1.0, 0.0)
            return 0
        lax.fori_loop(0, _S // 16, mk_, 0)

        def scat(i, _):
            mm = lists_v[pl.ds(k * _LW + i, 16)][0]
            idm = ids_v[pl.ds(mm, 16)][0]
            r = idm - nlo
            wt = wtab_v[pl.ds(idm - half0, 16)][0]

            @pl.when(wt == mm)
            def _():
                wv = mask_v[pl.ds(r, 16)]
                mask_v[pl.ds(r, 16)] = jnp.where(lane0, 1.0, wv)
                wv = tconf_v[pl.ds(r, 16)]
                tconf_v[pl.ds(r, 16)] = jnp.where(lane0, 1.0, wv)

                cc = clsi_v[pl.ds(mm, 16)][0]
                o = _C * r
                wv = tcls_v[pl.ds(o, 16)]
                tcls_v[pl.ds(o, 16)] = jnp.where(lane0, 0.0, wv)
                wv = tcls_v[pl.ds(o + cc, 16)]
                tcls_v[pl.ds(o + cc, 16)] = jnp.where(lane0, 1.0, wv)

                bw = boxes_v[pl.ds(4 * r, 16)]
                tb = lwhi_v[pl.ds(2 * mm, 16)] - _log_vec(bw)
                wv = tbox_v[pl.ds(4 * r + 2, 16)]
                tbox_v[pl.ds(4 * r + 2, 16)] = jnp.where(lane01, tb, wv)
            return 0
        lax.fori_loop(0, cnt, scat, 0)

        d1 = pltpu.async_copy(mask_v.at[pl.ds(0, _S)],
                              mask_hbm.at[b, pl.ds(nlo, _S)], sem_out)
        d2 = pltpu.async_copy(tconf_v.at[pl.ds(0, _S)],
                              tconf_hbm.at[b, pl.ds(nlo, _S)], sem_out)
        d3 = pltpu.async_copy(tcls_v.at[pl.ds(0, _C * _S)],
                              tcls_hbm.at[b, pl.ds(_C * nlo, _C * _S)],
                              sem_out)
        d4 = pltpu.async_copy(tbox_v.at[pl.ds(0, 4 * _S)],
                              tbox_hbm.at[b, pl.ds(4 * nlo, 4 * _S)],
                              sem_out)
        d1.wait()
        d2.wait()
        d3.wait()
        d4.wait()

        def rest(i, _):
            mm = lists_v[pl.ds(k * _LW + i, 16)][0]
            r = ids_v[pl.ds(mm, 16)][0] - nlo
            cc = clsi_v[pl.ds(mm, 16)][0]
            o = _C * r
            wv = tcls_v[pl.ds(o + cc, 16)]
            tcls_v[pl.ds(o + cc, 16)] = jnp.where(lane0, 0.0, wv)
            wv = tcls_v[pl.ds(o, 16)]
            tcls_v[pl.ds(o, 16)] = jnp.where(lane0, 1.0, wv)
            wv = tbox_v[pl.ds(4 * r + 2, 16)]
            tbox_v[pl.ds(4 * r + 2, 16)] = jnp.where(lane01, 0.0, wv)
            wv = tconf_v[pl.ds(r, 16)]
            tconf_v[pl.ds(r, 16)] = jnp.where(lane0, 0.0, wv)
            return 0
        lax.fori_loop(0, cnt, rest, 0)
        return 0

    lax.fori_loop(0, _K, chunk, 0)


def kernel(boxes, gt_boxes, match_pos_flag, match_gt_id):
    B, N, _ = boxes.shape
    _, M, _ = gt_boxes.shape
    C = _C

    x1y1 = gt_boxes[..., 0:2].reshape(B, 2 * M)
    x2y2 = gt_boxes[..., 2:4].reshape(B, 2 * M)
    clsi = gt_boxes[..., 4].astype(jnp.int32)

    sc_call = pl.kernel(
        _body,
        out_type=(
            jax.ShapeDtypeStruct((B, N), jnp.float32),
            jax.ShapeDtypeStruct((B, N), jnp.float32),
            jax.ShapeDtypeStruct((B, N * C), jnp.float32),
            jax.ShapeDtypeStruct((B, N * 4), jnp.float32),
        ),
        mesh=plsc.VectorSubcoreMesh(core_axis_name="c", subcore_axis_name="s"),
        compiler_params=pltpu.CompilerParams(use_tc_tiling_on_sc=True),
        scratch_types=[
            pltpu.VMEM((_M + 16,), jnp.int32),
            pltpu.VMEM((_M + 16,), jnp.int32),
            pltpu.VMEM((2 * _M + 16,), jnp.float32),
            pltpu.VMEM((2 * _M,), jnp.float32),
            pltpu.VMEM((2 * _M,), jnp.float32),
            pltpu.VMEM((_K * _LW + 16,), jnp.int32),
            pltpu.VMEM((_K + 32,), jnp.int32),
            pltpu.VMEM((_HALF + 16,), jnp.int32),
            pltpu.VMEM((_S,), jnp.int32),
            pltpu.VMEM((4 * _S + 16,), jnp.float32),
            pltpu.VMEM((_S + 16,), jnp.float32),
            pltpu.VMEM((_S + 16,), jnp.float32),
            pltpu.VMEM((_C * _S + 16,), jnp.float32),
            pltpu.VMEM((4 * _S + 16,), jnp.float32),
            pltpu.SemaphoreType.DMA,
            pltpu.SemaphoreType.DMA,
        ],
    )
    mask, tconf, tcls2, tbox2 = sc_call(boxes.reshape(B, N * 4), x1y1,
                                        x2y2, clsi, match_pos_flag,
                                        match_gt_id)
    return (mask, tconf, tcls2.reshape(B, N, C), tbox2.reshape(B, N, 4))


# speedup vs baseline: 6.0379x; 6.0379x over previous
import jax
import jax.numpy as jnp
from jax import lax
from jax.experimental import pallas as pl
from jax.experimental.pallas import tpu as pltpu
from jax.experimental.pallas import tpu_sc as plsc

_B, _N, _M, _C = 16, 16384, 128, 80
_NC, _NS = 2, 16
_HALF = _N // 2
_K = 16
_S = _HALF // _K
_LW = 160
_LN2 = 0.6931471805599453


def _log_vec(x):
    bits = lax.bitcast_convert_type(x, jnp.int32)
    e = (bits >> 23) - 127
    mant = lax.bitcast_convert_type((bits & 0x7FFFFF) | 0x3F800000,
                                    jnp.float32)
    t = (mant - 1.0) / (mant + 1.0)
    t2 = t * t
    p = 2.0 / 9.0
    p = p * t2 + 2.0 / 7.0
    p = p * t2 + 2.0 / 5.0
    p = p * t2 + 2.0 / 3.0
    p = p * t2 + 2.0
    return e.astype(jnp.float32) * _LN2 + t * p


def _body(boxes_hbm, x1y1_hbm, x2y2_hbm, cls_hbm, flag_hbm, ids_hbm,
          mask_hbm, tconf_hbm, tcls_hbm, tbox_hbm,
          ids_v, clsi_v, lwhi_v, gtp_v, gtq_v, lists_v, cnts_v, wtab_v,
          flag_v, boxes_v, mask_v, tconf_v, tcls_v, tbox_v,
          sem_in, sem_out):
    w = lax.axis_index("s") * _NC + lax.axis_index("c")
    b = w // 2
    h = w % 2
    half0 = h * _HALF

    pltpu.sync_copy(ids_hbm.at[b], ids_v.at[pl.ds(0, _M)])
    pltpu.sync_copy(cls_hbm.at[b], clsi_v.at[pl.ds(0, _M)])
    pltpu.sync_copy(x1y1_hbm.at[b], gtp_v)
    pltpu.sync_copy(x2y2_hbm.at[b], gtq_v)

    lane = lax.iota(jnp.int32, 16)
    zvec = jnp.zeros((16,), jnp.float32)
    zivec = jnp.zeros((16,), jnp.int32)
    e0vec = jnp.where(lane == 0, 1.0, 0.0)
    lane0 = lane == 0
    lane01 = lane < 2

    def prec(g, _):
        sl = pl.ds(g * 16, 16)
        lwhi_v[sl] = _log_vec(gtq_v[sl] - gtp_v[sl])
        return 0
    lax.fori_loop(0, 2 * _M // 16, prec, 0)

    cnts_v[pl.ds(0, 16)] = zivec
    cnts_v[pl.ds(16, 16)] = zivec

    def bldall(m, _):
        idm = ids_v[pl.ds(m, 16)][0]
        rr = idm - half0
        inw = (rr >= 0) & (rr < _HALF)

        @pl.when(inw)
        def _():
            kk = rr >> 9
            cv = cnts_v[pl.ds(kk, 16)][0]
            lv = lists_v[pl.ds(kk * _LW + cv, 16)]
            lists_v[pl.ds(kk * _LW + cv, 16)] = jnp.where(lane0, m, lv)
            cw = cnts_v[pl.ds(kk, 16)]
            cnts_v[pl.ds(kk, 16)] = jnp.where(lane0, cv + 1, cw)
            wv = wtab_v[pl.ds(rr, 16)]
            wtab_v[pl.ds(rr, 16)] = jnp.where(lane0, m, wv)
        return 0
    lax.fori_loop(0, _M, bldall, 0)

    dz = cnts_v[pl.ds(_K, 16)][0] * 0 + cnts_v[pl.ds(_K + 1, 16)][0] * 0

    def initr(r, _):
        tcls_v[r, pl.ds(0, 16)] = e0vec
        tcls_v[r, pl.ds(16, 16)] = zvec
        tcls_v[r, pl.ds(32, 16)] = zvec
        tcls_v[r, pl.ds(48, 16)] = zvec
        tcls_v[r, pl.ds(64, 16)] = zvec
        return 0
    lax.fori_loop(0, _S, initr, 0)

    def initc(g, _):
        tconf_v[pl.ds(g * 16, 16)] = zvec
        return 0
    lax.fori_loop(0, _S // 16, initc, 0)

    def initz(g, _):
        tbox_v[pl.ds(g * 16, 16)] = zvec
        return 0
    lax.fori_loop(0, 4 * _S // 16, initz, 0)

    def chunk(k, _):
        nlo = half0 + k * _S

        din1 = pltpu.async_copy(flag_hbm.at[b, pl.ds(nlo, _S)],
                                flag_v, sem_in)
        din2 = pltpu.async_copy(boxes_hbm.at[b, pl.ds(4 * nlo, 4 * _S)],
                                boxes_v.at[pl.ds(0, 4 * _S)], sem_in)
        cnt = cnts_v[pl.ds(k, 16)][0]

        din1.wait()
        din2.wait()

        def mk_(g, _):
            sl = pl.ds(g * 16, 16)
            mask_v[sl] = jnp.where(flag_v[sl] > 0, -1.0, 0.0)
            return 0
        lax.fori_loop(0, _S // 16, mk_, 0)

        def scat(i, _):
            mm = lists_v[pl.ds(k * _LW + i, 16)][0]
            idm = ids_v[pl.ds(mm, 16)][0]
            r = idm - nlo
            wt = wtab_v[pl.ds(idm - half0, 16)][0]

            @pl.when(wt == mm)
            def _():
                wv = mask_v[pl.ds(r, 16)]
                mask_v[pl.ds(r, 16)] = jnp.where(lane0, 1.0, wv)
                wv = tconf_v[pl.ds(r, 16)]
                tconf_v[pl.ds(r, 16)] = jnp.where(lane0, 1.0, wv)

                cc = clsi_v[pl.ds(mm, 16)][0]
                wv = tcls_v[r, pl.ds(dz, 16)]
                tcls_v[r, pl.ds(dz, 16)] = jnp.where(lane0, 0.0, wv)
                wv = tcls_v[r, pl.ds(cc, 16)]
                tcls_v[r, pl.ds(cc, 16)] = jnp.where(lane0, 1.0, wv)

                bw = boxes_v[pl.ds(4 * r, 16)]
                tb = lwhi_v[pl.ds(2 * mm, 16)] - _log_vec(bw)
                wv = tbox_v[pl.ds(4 * r + 2, 16)]
                tbox_v[pl.ds(4 * r + 2, 16)] = jnp.where(lane01, tb, wv)
            return 0
        lax.fori_loop(0, cnt, scat, 0)

        d1 = pltpu.async_copy(mask_v.at[pl.ds(0, _S)],
                              mask_hbm.at[b, pl.ds(nlo, _S)], sem_out)
        d2 = pltpu.async_copy(tconf_v.at[pl.ds(0, _S)],
                              tconf_hbm.at[b, pl.ds(nlo, _S)], sem_out)
        d3 = pltpu.async_copy(tcls_v.at[pl.ds(0, _S)],
                              tcls_hbm.at[b, pl.ds(nlo, _S)], sem_out)
        d4 = pltpu.async_copy(tbox_v.at[pl.ds(0, 4 * _S)],
                              tbox_hbm.at[b, pl.ds(4 * nlo, 4 * _S)],
                              sem_out)
        d1.wait()
        d2.wait()
        d3.wait()
        d4.wait()

        def rest(i, _):
            mm = lists_v[pl.ds(k * _LW + i, 16)][0]
            r = ids_v[pl.ds(mm, 16)][0] - nlo
            cc = clsi_v[pl.ds(mm, 16)][0]
            wv = tcls_v[r, pl.ds(cc, 16)]
            tcls_v[r, pl.ds(cc, 16)] = jnp.where(lane0, 0.0, wv)
            wv = tcls_v[r, pl.ds(dz, 16)]
            tcls_v[r, pl.ds(dz, 16)] = jnp.where(lane0, 1.0, wv)
            wv = tbox_v[pl.ds(4 * r + 2, 16)]
            tbox_v[pl.ds(4 * r + 2, 16)] = jnp.where(lane01, 0.0, wv)
            wv = tconf_v[pl.ds(r, 16)]
            tconf_v[pl.ds(r, 16)] = jnp.where(lane0, 0.0, wv)
            return 0
        lax.fori_loop(0, cnt, rest, 0)
        return 0

    lax.fori_loop(0, _K, chunk, 0)


def kernel(boxes, gt_boxes, match_pos_flag, match_gt_id):
    B, N, _ = boxes.shape
    _, M, _ = gt_boxes.shape
    C = _C

    x1y1 = gt_boxes[..., 0:2].reshape(B, 2 * M)
    x2y2 = gt_boxes[..., 2:4].reshape(B, 2 * M)
    clsi = gt_boxes[..., 4].astype(jnp.int32)

    sc_call = pl.kernel(
        _body,
        out_type=(
            jax.ShapeDtypeStruct((B, N), jnp.float32),
            jax.ShapeDtypeStruct((B, N), jnp.float32),
            jax.ShapeDtypeStruct((B, N, C), jnp.float32),
            jax.ShapeDtypeStruct((B, N * 4), jnp.float32),
        ),
        mesh=plsc.VectorSubcoreMesh(core_axis_name="c", subcore_axis_name="s"),
        compiler_params=pltpu.CompilerParams(use_tc_tiling_on_sc=True),
        scratch_types=[
            pltpu.VMEM((_M + 16,), jnp.int32),
            pltpu.VMEM((_M + 16,), jnp.int32),
            pltpu.VMEM((2 * _M + 16,), jnp.float32),
            pltpu.VMEM((2 * _M,), jnp.float32),
            pltpu.VMEM((2 * _M,), jnp.float32),
            pltpu.VMEM((_K * _LW + 16,), jnp.int32),
            pltpu.VMEM((_K + 32,), jnp.int32),
            pltpu.VMEM((_HALF + 16,), jnp.int32),
            pltpu.VMEM((_S,), jnp.int32),
            pltpu.VMEM((4 * _S + 16,), jnp.float32),
            pltpu.VMEM((_S + 16,), jnp.float32),
            pltpu.VMEM((_S + 16,), jnp.float32),
            pltpu.VMEM((_S, _C), jnp.float32),
            pltpu.VMEM((4 * _S + 16,), jnp.float32),
            pltpu.SemaphoreType.DMA,
            pltpu.SemaphoreType.DMA,
        ],
    )
    mask, tconf, tcls, tbox2 = sc_call(boxes.reshape(B, N * 4), x1y1,
                                       x2y2, clsi, match_pos_flag,
                                       match_gt_id)
    return (mask, tconf, tcls, tbox2.reshape(B, N, 4))
